# baseline (speedup 1.0000x reference)
import functools

import jax
import jax.numpy as jnp
from jax.experimental import pallas as pl
from jax.experimental.pallas import tpu as pltpu

_K = 8
_IBIG = jnp.iinfo(jnp.int32).max
_CMLANES = 256


def _phase_a_body(dist_ref, t_ref, cm_ref, *, n_cols, block_c):
    j = pl.program_id(0)
    nblk = pl.num_programs(0)
    rows = dist_ref.shape[0]

    @pl.when(j == 0)
    def _init():
        cm_ref[...] = jnp.full((_CMLANES, rows), jnp.inf, jnp.float32)

    col_row = j * block_c + jax.lax.broadcasted_iota(
        jnp.int32, (1, block_c), 1)
    d = jnp.where(col_row < n_cols, dist_ref[...], jnp.inf)
    cm_ref[pl.ds(j, 1), :] = jnp.min(d, axis=1)[None, :]

    @pl.when(j == nblk - 1)
    def _threshold():
        cm = cm_ref[...]
        mv = jnp.min(cm, axis=0)
        for _ in range(_K - 1):
            cm = jnp.where(cm == mv[None, :], jnp.inf, cm)
            mv = jnp.min(cm, axis=0)
        t_ref[...] = mv


def _phase_b_body(dist_ref, labels_ref, t_ref, out_ref,
                  accv_ref, acci_ref, accl_ref, *, n_cols, block_c):
    j = pl.program_id(0)
    nblk = pl.num_programs(0)
    rows = dist_ref.shape[0]

    @pl.when(j == 0)
    def _init():
        accv_ref[...] = jnp.full((_K, rows), jnp.inf, jnp.float32)
        acci_ref[...] = jnp.full((_K, rows), _IBIG, jnp.int32)
        accl_ref[...] = jnp.full((_K, rows), _IBIG, jnp.int32)

    col0 = j * block_c
    col_row = col0 + jax.lax.broadcasted_iota(jnp.int32, (1, block_c), 1)
    d = jnp.where(col_row < n_cols, dist_ref[...], jnp.inf)
    lrow = labels_ref[:, pl.ds(col0, block_c)]
    tvec = t_ref[...]

    def cond(carry):
        _, bm, _, _, _ = carry
        return jnp.any(bm <= tvec)

    def body(carry):
        d, bm, accv, acci, accl = carry
        m = bm
        is_min = d == m[:, None]
        idx = jnp.min(jnp.where(is_min, col_row, _IBIG), axis=1)
        at = col_row == idx[:, None]
        lbl = jnp.min(jnp.where(at, lrow, _IBIG), axis=1)
        accept = m <= tvec
        cv = jnp.where(accept, m, jnp.inf)
        ci = jnp.where(accept, idx, _IBIG)
        cl = jnp.where(accept, lbl, _IBIG)
        nv, ni, nl = [], [], []
        for s in range(_K):
            av, ai, al = accv[s], acci[s], accl[s]
            lt = (cv < av) | ((cv == av) & (ci < ai))
            nv.append(jnp.where(lt, cv, av))
            ni.append(jnp.where(lt, ci, ai))
            nl.append(jnp.where(lt, cl, al))
            cv = jnp.where(lt, av, cv)
            ci = jnp.where(lt, ai, ci)
            cl = jnp.where(lt, al, cl)
        accv = jnp.stack(nv)
        acci = jnp.stack(ni)
        accl = jnp.stack(nl)
        d = jnp.where(at, jnp.inf, d)
        bm = jnp.min(d, axis=1)
        return d, bm, accv, acci, accl

    bm0 = jnp.min(d, axis=1)
    carry = (d, bm0, accv_ref[...], acci_ref[...], accl_ref[...])
    _, _, accv, acci, accl = jax.lax.while_loop(cond, body, carry)
    accv_ref[...] = accv
    acci_ref[...] = acci
    accl_ref[...] = accl

    @pl.when(j == nblk - 1)
    def _vote():
        lab = [accl[s] for s in range(_K)]
        cnt = []
        for i in range(_K):
            c = jnp.ones((rows,), jnp.int32)
            for jj in range(_K):
                if jj != i:
                    c = c + (lab[i] == lab[jj]).astype(jnp.int32)
            cnt.append(c)
        maxc = cnt[0]
        for i in range(1, _K):
            maxc = jnp.maximum(maxc, cnt[i])
        pred = jnp.full((rows,), _IBIG, jnp.int32)
        for i in range(_K):
            pred = jnp.minimum(pred, jnp.where(cnt[i] == maxc, lab[i], _IBIG))
        out_ref[...] = pred


@jax.jit
def kernel(distances, labels):
    rows, n_cols = distances.shape
    block_c = 512
    nblk = pl.cdiv(n_cols, block_c)
    assert nblk <= _CMLANES
    n_pad = nblk * block_c
    labels2d = jnp.pad(labels, (0, n_pad - n_cols)).reshape(1, n_pad)

    t = pl.pallas_call(
        functools.partial(_phase_a_body, n_cols=n_cols, block_c=block_c),
        grid=(nblk,),
        in_specs=[pl.BlockSpec((rows, block_c), lambda j: (0, j))],
        out_specs=pl.BlockSpec((rows,), lambda j: (0,)),
        out_shape=jax.ShapeDtypeStruct((rows,), jnp.float32),
        scratch_shapes=[pltpu.VMEM((_CMLANES, rows), jnp.float32)],
    )(distances)

    return pl.pallas_call(
        functools.partial(_phase_b_body, n_cols=n_cols, block_c=block_c),
        grid=(nblk,),
        in_specs=[
            pl.BlockSpec((rows, block_c), lambda j: (0, j)),
            pl.BlockSpec((1, n_pad), lambda j: (0, 0)),
            pl.BlockSpec((rows,), lambda j: (0,)),
        ],
        out_specs=pl.BlockSpec((rows,), lambda j: (0,)),
        out_shape=jax.ShapeDtypeStruct((rows,), jnp.int32),
        scratch_shapes=[
            pltpu.VMEM((_K, rows), jnp.float32),
            pltpu.VMEM((_K, rows), jnp.int32),
            pltpu.VMEM((_K, rows), jnp.int32),
        ],
    )(distances, labels2d, t)


# speedup vs baseline: 1.0103x; 1.0103x over previous
import functools

import jax
import jax.numpy as jnp
from jax.experimental import pallas as pl
from jax.experimental.pallas import tpu as pltpu

_K = 8
_IBIG = jnp.iinfo(jnp.int32).max


def _phase_a_body(dist_ref, t_ref, m_ref, *, n_cols, block_c):
    j = pl.program_id(0)
    nblk = pl.num_programs(0)
    rows = dist_ref.shape[0]

    @pl.when(j == 0)
    def _init():
        m_ref[...] = jnp.full((rows, 128), jnp.inf, jnp.float32)

    col_row = j * block_c + jax.lax.broadcasted_iota(
        jnp.int32, (1, block_c), 1)
    d = jnp.where(col_row < n_cols, dist_ref[...], jnp.inf)
    m = m_ref[...]
    for s in range(block_c // 128):
        m = jnp.minimum(m, d[:, s * 128:(s + 1) * 128])
    m_ref[...] = m

    @pl.when(j == nblk - 1)
    def _threshold():
        cm = m_ref[...]
        mv = jnp.min(cm, axis=1)
        for _ in range(_K - 1):
            cm = jnp.where(cm == mv[:, None], jnp.inf, cm)
            mv = jnp.min(cm, axis=1)
        t_ref[...] = mv


def _phase_b_body(dist_ref, labels_ref, t_ref, out_ref,
                  accv_ref, acci_ref, accl_ref, *, n_cols, block_c):
    j = pl.program_id(0)
    nblk = pl.num_programs(0)
    rows = dist_ref.shape[0]

    @pl.when(j == 0)
    def _init():
        accv_ref[...] = jnp.full((_K, rows), jnp.inf, jnp.float32)
        acci_ref[...] = jnp.full((_K, rows), _IBIG, jnp.int32)
        accl_ref[...] = jnp.full((_K, rows), _IBIG, jnp.int32)

    col0 = j * block_c
    col_row = col0 + jax.lax.broadcasted_iota(jnp.int32, (1, block_c), 1)
    d = jnp.where(col_row < n_cols, dist_ref[...], jnp.float32(2.0))
    lrow = labels_ref[:, pl.ds(col0, block_c)]
    tvec = t_ref[...]

    cand = d <= tvec[:, None]
    w0 = jnp.where(cand, col_row, _IBIG)

    def cond(carry):
        _, idxsel, _, _, _ = carry
        return jnp.min(idxsel) < _IBIG

    def body(carry):
        w, idxsel, accv, acci, accl = carry
        at = w == idxsel[:, None]
        val = jnp.min(jnp.where(at, d, jnp.inf), axis=1)
        lbl = jnp.min(jnp.where(at, lrow, _IBIG), axis=1)
        accept = idxsel < _IBIG
        cv = jnp.where(accept, val, jnp.inf)
        ci = jnp.where(accept, idxsel, _IBIG)
        cl = jnp.where(accept, lbl, _IBIG)
        nv, ni, nl = [], [], []
        for s in range(_K):
            av, ai, al = accv[s], acci[s], accl[s]
            lt = (cv < av) | ((cv == av) & (ci < ai))
            nv.append(jnp.where(lt, cv, av))
            ni.append(jnp.where(lt, ci, ai))
            nl.append(jnp.where(lt, cl, al))
            cv = jnp.where(lt, av, cv)
            ci = jnp.where(lt, ai, ci)
            cl = jnp.where(lt, al, cl)
        accv = jnp.stack(nv)
        acci = jnp.stack(ni)
        accl = jnp.stack(nl)
        w = jnp.where(at, _IBIG, w)
        idxsel = jnp.min(w, axis=1)
        return w, idxsel, accv, acci, accl

    carry = (w0, jnp.min(w0, axis=1),
             accv_ref[...], acci_ref[...], accl_ref[...])
    _, _, accv, acci, accl = jax.lax.while_loop(cond, body, carry)
    accv_ref[...] = accv
    acci_ref[...] = acci
    accl_ref[...] = accl

    @pl.when(j == nblk - 1)
    def _vote():
        lab = [accl[s] for s in range(_K)]
        cnt = []
        for i in range(_K):
            c = jnp.ones((rows,), jnp.int32)
            for jj in range(_K):
                if jj != i:
                    c = c + (lab[i] == lab[jj]).astype(jnp.int32)
            cnt.append(c)
        maxc = cnt[0]
        for i in range(1, _K):
            maxc = jnp.maximum(maxc, cnt[i])
        pred = jnp.full((rows,), _IBIG, jnp.int32)
        for i in range(_K):
            pred = jnp.minimum(pred, jnp.where(cnt[i] == maxc, lab[i], _IBIG))
        out_ref[...] = pred


@jax.jit
def kernel(distances, labels):
    rows, n_cols = distances.shape
    block_c = 512
    nblk = pl.cdiv(n_cols, block_c)
    n_pad = nblk * block_c
    labels2d = jnp.pad(labels, (0, n_pad - n_cols)).reshape(1, n_pad)

    t = pl.pallas_call(
        functools.partial(_phase_a_body, n_cols=n_cols, block_c=block_c),
        grid=(nblk,),
        in_specs=[pl.BlockSpec((rows, block_c), lambda j: (0, j))],
        out_specs=pl.BlockSpec((rows,), lambda j: (0,)),
        out_shape=jax.ShapeDtypeStruct((rows,), jnp.float32),
        scratch_shapes=[pltpu.VMEM((rows, 128), jnp.float32)],
    )(distances)

    return pl.pallas_call(
        functools.partial(_phase_b_body, n_cols=n_cols, block_c=block_c),
        grid=(nblk,),
        in_specs=[
            pl.BlockSpec((rows, block_c), lambda j: (0, j)),
            pl.BlockSpec((1, n_pad), lambda j: (0, 0)),
            pl.BlockSpec((rows,), lambda j: (0,)),
        ],
        out_specs=pl.BlockSpec((rows,), lambda j: (0,)),
        out_shape=jax.ShapeDtypeStruct((rows,), jnp.int32),
        scratch_shapes=[
            pltpu.VMEM((_K, rows), jnp.float32),
            pltpu.VMEM((_K, rows), jnp.int32),
            pltpu.VMEM((_K, rows), jnp.int32),
        ],
    )(distances, labels2d, t)


# speedup vs baseline: 1.1724x; 1.1605x over previous
import functools

import jax
import jax.numpy as jnp
from jax.experimental import pallas as pl
from jax.experimental.pallas import tpu as pltpu

_K = 8
_IBIG = jnp.iinfo(jnp.int32).max


def _knnc_body(dist_ref, labels_ref, out_ref, acc_val, acc_idx, acc_lbl,
               *, n_cols, block_c):
    j = pl.program_id(0)
    nblk = pl.num_programs(0)
    rows = dist_ref.shape[0]

    @pl.when(j == 0)
    def _init():
        acc_val[...] = jnp.full((rows, _K), jnp.inf, jnp.float32)
        acc_idx[...] = jnp.full((rows, _K), _IBIG, jnp.int32)
        acc_lbl[...] = jnp.full((rows, _K), _IBIG, jnp.int32)

    col0 = j * block_c
    colid = col0 + jax.lax.broadcasted_iota(jnp.int32, (rows, block_c), 1)
    d = dist_ref[...]
    d = jnp.where(colid < n_cols, d, jnp.inf)
    lrow = labels_ref[:, pl.ds(col0, block_c)]

    bv, bi, bl = [], [], []
    for _ in range(_K):
        m = jnp.min(d, axis=1)
        is_min = d == m[:, None]
        idx = jnp.min(jnp.where(is_min, colid, _IBIG), axis=1)
        at = colid == idx[:, None]
        lbl = jnp.min(jnp.where(at, lrow, _IBIG), axis=1)
        d = jnp.where(at, jnp.inf, d)
        bv.append(m)
        bi.append(idx)
        bl.append(lbl)

    allv = jnp.concatenate([acc_val[...]] + [v[:, None] for v in bv], axis=1)
    alli = jnp.concatenate([acc_idx[...]] + [v[:, None] for v in bi], axis=1)
    alll = jnp.concatenate([acc_lbl[...]] + [v[:, None] for v in bl], axis=1)

    ov, oi, ol = [], [], []
    for _ in range(_K):
        m = jnp.min(allv, axis=1)
        is_min = allv == m[:, None]
        idx = jnp.min(jnp.where(is_min, alli, _IBIG), axis=1)
        sel = is_min & (alli == idx[:, None])
        lbl = jnp.min(jnp.where(sel, alll, _IBIG), axis=1)
        allv = jnp.where(sel, jnp.inf, allv)
        ov.append(m)
        oi.append(idx)
        ol.append(lbl)
    acc_val[...] = jnp.concatenate([v[:, None] for v in ov], axis=1)
    acc_idx[...] = jnp.concatenate([v[:, None] for v in oi], axis=1)
    acc_lbl[...] = jnp.concatenate([v[:, None] for v in ol], axis=1)

    @pl.when(j == nblk - 1)
    def _vote():
        lab = acc_lbl[...]
        cnt = jnp.ones((rows, _K), jnp.int32)
        for s in range(1, _K):
            rolled = jnp.concatenate([lab[:, s:], lab[:, :s]], axis=1)
            cnt = cnt + (lab == rolled).astype(jnp.int32)
        maxc = jnp.max(cnt, axis=1)
        masked = jnp.where(cnt == maxc[:, None], lab, _IBIG)
        out_ref[...] = jnp.min(masked, axis=1)


@jax.jit
def kernel(distances, labels):
    rows, n_cols = distances.shape
    block_c = 512
    nblk = pl.cdiv(n_cols, block_c)
    n_pad = nblk * block_c
    labels2d = jnp.pad(labels, (0, n_pad - n_cols)).reshape(1, n_pad)
    body = functools.partial(_knnc_body, n_cols=n_cols, block_c=block_c)
    return pl.pallas_call(
        body,
        grid=(nblk,),
        in_specs=[
            pl.BlockSpec((rows, block_c), lambda j: (0, j)),
            pl.BlockSpec((1, n_pad), lambda j: (0, 0)),
        ],
        out_specs=pl.BlockSpec((rows,), lambda j: (0,)),
        out_shape=jax.ShapeDtypeStruct((rows,), jnp.int32),
        scratch_shapes=[
            pltpu.VMEM((rows, _K), jnp.float32),
            pltpu.VMEM((rows, _K), jnp.int32),
            pltpu.VMEM((rows, _K), jnp.int32),
        ],
    )(distances, labels2d)
